# baseline (speedup 1.0000x reference)
import functools

import jax
import jax.numpy as jnp
from jax import lax
from jax.experimental import pallas as pl
from jax.experimental.pallas import tpu as pltpu
from jax.experimental.pallas import tpu_sc as plsc

MAXL = 512
NH = 16
QLEN = 2048
KLEN = 2048
EW = 4096


@functools.partial(
    pl.kernel,
    out_type=jax.ShapeDtypeStruct((NH * QLEN * KLEN,), jnp.float32),
    mesh=plsc.VectorSubcoreMesh(core_axis_name="c", subcore_axis_name="s"),
    scratch_types=[pltpu.VMEM((EW,), jnp.float32) for _ in range(8)]
    + [pltpu.SemaphoreType.DMA],
)
def _sc_bias(e8_hbm, out_hbm, *scratch):
    es, sem = scratch[:8], scratch[8]
    h = lax.axis_index("s")
    half = lax.axis_index("c")
    for s in range(8):
        pltpu.sync_copy(e8_hbm.at[pl.ds((h * 8 + s) * EW, EW)], es[s])
    i0_half = half * 1024
    row0 = h * QLEN + i0_half

    LAG = 4

    def drain_one_group():
        for _ in range(8):
            pltpu.make_async_copy(
                es[0].at[pl.ds(0, KLEN)],
                out_hbm.at[pl.ds(0, KLEN)],
                sem,
            ).wait()

    def group(b, carry):
        base = 2040 - i0_half - 8 * b
        r0 = row0 + 8 * b
        for u in range(8):
            pltpu.async_copy(
                es[7 - u].at[pl.ds(base, KLEN)],
                out_hbm.at[pl.ds((r0 + u) * KLEN, KLEN)],
                sem,
            )
        pl.when(b >= LAG)(drain_one_group)
        return carry

    lax.fori_loop(0, 128, group, 0)
    for _ in range(LAG):
        drain_one_group()


def kernel(qlen, klen, rel):
    del qlen, klen
    rel = rel.astype(jnp.float32)
    ef = jnp.concatenate(
        [
            jnp.broadcast_to(rel[0:1], (QLEN - MAXL - 1, NH)),
            rel,
            jnp.broadcast_to(rel[2 * MAXL : 2 * MAXL + 1], (KLEN - MAXL + 8, NH)),
        ],
        axis=0,
    ).T
    e8 = jnp.stack(
        [lax.slice_in_dim(ef, s, s + EW, axis=1) for s in range(8)], axis=1
    )
    return _sc_bias(e8.reshape(-1)).reshape(NH, QLEN, KLEN)
